# initial kernel scaffold (unmeasured)
import jax
import jax.numpy as jnp
from jax import lax
from jax.experimental import pallas as pl
from jax.experimental.pallas import tpu as pltpu


def kernel(
    x,
):
    def body(*refs):
        pass

    out_shape = jax.ShapeDtypeStruct(..., jnp.float32)
    return pl.pallas_call(body, out_shape=out_shape)(...)


# baseline (device time: 28000 ns/iter reference)
import functools

import jax
import jax.numpy as jnp
from jax import lax
from jax.experimental import pallas as pl
from jax.experimental.pallas import tpu as pltpu

N_DEV = 16
LOG2_N = 4


def kernel(x):
    m, n = x.shape

    def body(x_ref, out_ref, recv_buf, send_sems, recv_sems):
        my = lax.axis_index("i")

        barrier_sem = pltpu.get_barrier_semaphore()
        for k in range(LOG2_N):
            partner = my ^ (1 << k)
            pl.semaphore_signal(
                barrier_sem, inc=1,
                device_id=(partner,), device_id_type=pl.DeviceIdType.MESH,
            )
        pl.semaphore_wait(barrier_sem, LOG2_N)

        out_ref[...] = x_ref[...]

        for k in range(LOG2_N):
            partner = my ^ (1 << k)
            rdma = pltpu.make_async_remote_copy(
                src_ref=out_ref,
                dst_ref=recv_buf.at[k],
                send_sem=send_sems.at[k],
                recv_sem=recv_sems.at[k],
                device_id=(partner,),
                device_id_type=pl.DeviceIdType.MESH,
            )
            rdma.start()
            rdma.wait()
            out_ref[...] = out_ref[...] + recv_buf[k]

    return pl.pallas_call(
        body,
        out_shape=jax.ShapeDtypeStruct((m, n), jnp.float32),
        in_specs=[pl.BlockSpec(memory_space=pltpu.VMEM)],
        out_specs=pl.BlockSpec(memory_space=pltpu.VMEM),
        scratch_shapes=[
            pltpu.VMEM((LOG2_N, m, n), jnp.float32),
            pltpu.SemaphoreType.DMA((LOG2_N,)),
            pltpu.SemaphoreType.DMA((LOG2_N,)),
        ],
        compiler_params=pltpu.CompilerParams(collective_id=0),
    )(x)


# device time: 17811 ns/iter; 1.5721x vs baseline; 1.5721x over previous
import jax
import jax.numpy as jnp
from jax import lax
from jax.experimental import pallas as pl
from jax.experimental.pallas import tpu as pltpu

N_DEV = 16


def kernel(x):
    m, n = x.shape
    S = m // N_DEV

    def body(x_ref, out_ref, rs_buf, rs_send, rs_recv, ag_send, ag_recv):
        my = lax.axis_index("i")

        barrier_sem = pltpu.get_barrier_semaphore()
        for d in range(1, N_DEV):
            peer = (my + d) % N_DEV
            pl.semaphore_signal(
                barrier_sem, inc=1,
                device_id=(peer,), device_id_type=pl.DeviceIdType.MESH,
            )
        pl.semaphore_wait(barrier_sem, N_DEV - 1)

        rs_sends = []
        for d in range(1, N_DEV):
            peer = (my + d) % N_DEV
            rdma = pltpu.make_async_remote_copy(
                src_ref=x_ref.at[pl.ds(peer * S, S)],
                dst_ref=rs_buf.at[my],
                send_sem=rs_send.at[peer],
                recv_sem=rs_recv.at[my],
                device_id=(peer,),
                device_id_type=pl.DeviceIdType.MESH,
            )
            rdma.start()
            rs_sends.append(rdma)

        rs_buf[my] = x_ref[pl.ds(my * S, S)]

        for d in range(1, N_DEV):
            j = (my + d) % N_DEV
            recv = pltpu.make_async_remote_copy(
                src_ref=x_ref.at[pl.ds(j * S, S)],
                dst_ref=rs_buf.at[j],
                send_sem=rs_send.at[j],
                recv_sem=rs_recv.at[j],
                device_id=(j,),
                device_id_type=pl.DeviceIdType.MESH,
            )
            recv.wait_recv()

        out_ref[pl.ds(my * S, S)] = jnp.sum(rs_buf[...], axis=0)

        ag_sends = []
        for d in range(1, N_DEV):
            peer = (my + d) % N_DEV
            rdma = pltpu.make_async_remote_copy(
                src_ref=out_ref.at[pl.ds(my * S, S)],
                dst_ref=out_ref.at[pl.ds(my * S, S)],
                send_sem=ag_send.at[peer],
                recv_sem=ag_recv.at[my],
                device_id=(peer,),
                device_id_type=pl.DeviceIdType.MESH,
            )
            rdma.start()
            ag_sends.append(rdma)

        for d in range(1, N_DEV):
            j = (my + d) % N_DEV
            recv = pltpu.make_async_remote_copy(
                src_ref=out_ref.at[pl.ds(j * S, S)],
                dst_ref=out_ref.at[pl.ds(j * S, S)],
                send_sem=ag_send.at[j],
                recv_sem=ag_recv.at[j],
                device_id=(j,),
                device_id_type=pl.DeviceIdType.MESH,
            )
            recv.wait_recv()

        for rdma in rs_sends:
            rdma.wait_send()
        for rdma in ag_sends:
            rdma.wait_send()

    return pl.pallas_call(
        body,
        out_shape=jax.ShapeDtypeStruct((m, n), jnp.float32),
        in_specs=[pl.BlockSpec(memory_space=pltpu.VMEM)],
        out_specs=pl.BlockSpec(memory_space=pltpu.VMEM),
        scratch_shapes=[
            pltpu.VMEM((N_DEV, S, n), jnp.float32),
            pltpu.SemaphoreType.DMA((N_DEV,)),
            pltpu.SemaphoreType.DMA((N_DEV,)),
            pltpu.SemaphoreType.DMA((N_DEV,)),
            pltpu.SemaphoreType.DMA((N_DEV,)),
        ],
        compiler_params=pltpu.CompilerParams(collective_id=0),
    )(x)


# device time: 17693 ns/iter; 1.5825x vs baseline; 1.0067x over previous
import jax
import jax.numpy as jnp
from jax import lax
from jax.experimental import pallas as pl
from jax.experimental.pallas import tpu as pltpu

N_DEV = 16

_W_XY = {0: (0, 0), 1: (1, 0), 2: (1, 1), 3: (0, 1)}


def _dist_from_0(d: int) -> int:
    x, y = _W_XY[d % 4]
    return d // 4 + x + y


_FAR_FIRST = sorted(range(1, N_DEV), key=_dist_from_0, reverse=True)


def kernel(x):
    m, n = x.shape
    S = m // N_DEV

    def body(x_ref, out_ref, rs_buf, rs_send, rs_recv, ag_send, ag_recv):
        my = lax.axis_index("i")

        barrier_sem = pltpu.get_barrier_semaphore()
        for d in range(1, N_DEV):
            peer = (my + d) % N_DEV
            pl.semaphore_signal(
                barrier_sem, inc=1,
                device_id=(peer,), device_id_type=pl.DeviceIdType.MESH,
            )
        pl.semaphore_wait(barrier_sem, N_DEV - 1)

        rs_sends = []
        for d in _FAR_FIRST:
            peer = (my + d) % N_DEV
            rdma = pltpu.make_async_remote_copy(
                src_ref=x_ref.at[pl.ds(peer * S, S)],
                dst_ref=rs_buf.at[my],
                send_sem=rs_send.at[peer],
                recv_sem=rs_recv.at[my],
                device_id=(peer,),
                device_id_type=pl.DeviceIdType.MESH,
            )
            rdma.start()
            rs_sends.append(rdma)

        acc = x_ref[pl.ds(my * S, S)]
        for d in range(1, N_DEV):
            j = (my + d) % N_DEV
            recv = pltpu.make_async_remote_copy(
                src_ref=x_ref.at[pl.ds(j * S, S)],
                dst_ref=rs_buf.at[j],
                send_sem=rs_send.at[j],
                recv_sem=rs_recv.at[j],
                device_id=(j,),
                device_id_type=pl.DeviceIdType.MESH,
            )
            recv.wait_recv()
            acc = acc + rs_buf[j]

        out_ref[pl.ds(my * S, S)] = acc

        ag_sends = []
        for d in _FAR_FIRST:
            peer = (my + d) % N_DEV
            rdma = pltpu.make_async_remote_copy(
                src_ref=out_ref.at[pl.ds(my * S, S)],
                dst_ref=out_ref.at[pl.ds(my * S, S)],
                send_sem=ag_send.at[peer],
                recv_sem=ag_recv.at[my],
                device_id=(peer,),
                device_id_type=pl.DeviceIdType.MESH,
            )
            rdma.start()
            ag_sends.append(rdma)

        for d in range(1, N_DEV):
            j = (my + d) % N_DEV
            recv = pltpu.make_async_remote_copy(
                src_ref=out_ref.at[pl.ds(j * S, S)],
                dst_ref=out_ref.at[pl.ds(j * S, S)],
                send_sem=ag_send.at[j],
                recv_sem=ag_recv.at[j],
                device_id=(j,),
                device_id_type=pl.DeviceIdType.MESH,
            )
            recv.wait_recv()

        for rdma in rs_sends:
            rdma.wait_send()
        for rdma in ag_sends:
            rdma.wait_send()

    return pl.pallas_call(
        body,
        out_shape=jax.ShapeDtypeStruct((m, n), jnp.float32),
        in_specs=[pl.BlockSpec(memory_space=pltpu.VMEM)],
        out_specs=pl.BlockSpec(memory_space=pltpu.VMEM),
        scratch_shapes=[
            pltpu.VMEM((N_DEV, S, n), jnp.float32),
            pltpu.SemaphoreType.DMA((N_DEV,)),
            pltpu.SemaphoreType.DMA((N_DEV,)),
            pltpu.SemaphoreType.DMA((N_DEV,)),
            pltpu.SemaphoreType.DMA((N_DEV,)),
        ],
        compiler_params=pltpu.CompilerParams(collective_id=0),
    )(x)
